# baseline (speedup 1.0000x reference)
import functools

import jax
import jax.numpy as jnp
from jax import lax
from jax.experimental import pallas as pl
from jax.experimental.pallas import tpu as pltpu
from jax.experimental.pallas import tpu_sc as plsc

N = 10000
E = 10000
NNZ = 320000
D = 128
NP = 10240
NC = 2
NS = 16
NW = NC * NS
PER_W = NNZ // NW
COLS = D // NS
SEG_K = 128
SEG_SUP = 10
SEG_NSUP = (NNZ // NC) // (SEG_K * SEG_SUP)

_f32 = jnp.float32
_i32 = jnp.int32

_MESH = plsc.VectorSubcoreMesh(core_axis_name="c", subcore_axis_name="s")


@functools.partial(
    pl.kernel,
    out_type=(
        jax.ShapeDtypeStruct((NW, NP // 16, 16), _f32),
        jax.ShapeDtypeStruct((NW, NP // 16, 16), _f32),
    ),
    mesh=_MESH,
    scratch_types=[
        pltpu.VMEM((PER_W,), _i32),
        pltpu.VMEM((PER_W,), _i32),
        pltpu.VMEM((E,), _f32),
        pltpu.VMEM((NP // 16, 16), _f32),
        pltpu.VMEM((NP // 16, 16), _f32),
    ],
    compiler_params=pltpu.CompilerParams(needs_layout_passes=False,
                                         use_tc_tiling_on_sc=False),
)
def _deg_kernel(nidx_hbm, eidx_hbm, hw_hbm, dg_out, bg_out,
                nidx_v, eidx_v, hw_v, dg_v, bg_v):
    cid = lax.axis_index("c")
    sid = lax.axis_index("s")
    wid = cid * NS + sid
    base = wid * PER_W
    pltpu.sync_copy(nidx_hbm.at[pl.ds(base, PER_W)], nidx_v)
    pltpu.sync_copy(eidx_hbm.at[pl.ds(base, PER_W)], eidx_v)
    pltpu.sync_copy(hw_hbm, hw_v)

    zeros16 = jnp.zeros((16,), _f32)

    def zero_body(i, carry):
        dg_v[i, :] = zeros16
        bg_v[i, :] = zeros16
        return carry

    lax.fori_loop(0, NP // 16, zero_body, 0)

    ones16 = jnp.ones((16,), _f32)

    def acc_body(i, carry):
        n16 = nidx_v[pl.ds(i * 16, 16)]
        e16 = eidx_v[pl.ds(i * 16, 16)]
        w16 = plsc.load_gather(hw_v, [e16])
        plsc.addupdate_scatter(dg_v, [n16 >> 4, n16 & 15], w16)
        plsc.addupdate_scatter(bg_v, [e16 >> 4, e16 & 15], ones16)
        return carry

    lax.fori_loop(0, PER_W // 16, acc_body, 0)

    pltpu.sync_copy(dg_v, dg_out.at[wid])
    pltpu.sync_copy(bg_v, bg_out.at[wid])


@functools.partial(
    pl.kernel,
    out_type=jax.ShapeDtypeStruct((NC, NS, NP, COLS), _f32),
    mesh=_MESH,
    scratch_types=[
        pltpu.VMEM((SEG_SUP, SEG_K), _i32),
        pltpu.VMEM((SEG_SUP, SEG_K), _i32),
        pltpu.VMEM((SEG_K, COLS), _f32),
        pltpu.VMEM((SEG_K, COLS), _f32),
        pltpu.VMEM((NP, COLS), _f32),
        pltpu.SemaphoreType.DMA,
        pltpu.SemaphoreType.DMA,
    ],
    compiler_params=pltpu.CompilerParams(needs_layout_passes=False,
                                         use_tc_tiling_on_sc=False),
)
def _seg_kernel(feat_hbm, gidx_hbm, sidx_hbm, zeros_hbm, out_hbm,
                gidx_v, sidx_v, rows_a, rows_b, acc_v, sem_a, sem_b):
    cid = lax.axis_index("c")
    sid = lax.axis_index("s")
    pltpu.sync_copy(zeros_hbm, acc_v)

    lane = lax.iota(_i32, 16)
    col_sel = lane & 7
    half = lane >> 3

    def accumulate(buf, j):
        jvec = jnp.zeros((16,), _i32) + j
        for g in range(SEG_K // 2):
            pair_sel = half + (2 * g)
            sel = plsc.load_gather(sidx_v, [jvec, pair_sel])
            vals = plsc.load_gather(buf, [pair_sel, col_sel])
            plsc.addupdate_scatter(acc_v, [sel, col_sel], vals)

    def pair_body(s, p, carry):
        j = p * 2
        da = pltpu.async_copy(feat_hbm.at[sid].at[gidx_v.at[j]],
                              rows_a, sem_a)
        db = pltpu.async_copy(feat_hbm.at[sid].at[gidx_v.at[j + 1]],
                              rows_b, sem_b)
        da.wait()
        accumulate(rows_a, j)
        db.wait()
        accumulate(rows_b, j + 1)
        return carry

    def sup_body(s, carry):
        pltpu.sync_copy(gidx_hbm.at[cid, s], gidx_v)
        pltpu.sync_copy(sidx_hbm.at[cid, s], sidx_v)
        return lax.fori_loop(0, SEG_SUP // 2,
                             functools.partial(pair_body, s), carry)

    lax.fori_loop(0, SEG_NSUP, sup_body, 0)
    pltpu.sync_copy(acc_v, out_hbm.at[cid, sid])


_BM = 2048


def _mm_body(x_ref, w_ref, o_ref):
    o_ref[...] = jnp.dot(x_ref[...], w_ref[...], preferred_element_type=_f32)


def _mm(xp, w):
    return pl.pallas_call(
        _mm_body,
        grid=(NP // _BM,),
        in_specs=[
            pl.BlockSpec((_BM, D), lambda i: (i, 0)),
            pl.BlockSpec((D, D), lambda i: (0, 0)),
        ],
        out_specs=pl.BlockSpec((_BM, D), lambda i: (i, 0)),
        out_shape=jax.ShapeDtypeStruct((NP, D), _f32),
    )(xp, w)


def _fuse_edge_body(p_ref, bgp_ref, o_ref):
    s = p_ref[0] + p_ref[1]
    bg = jnp.sum(bgp_ref[...], axis=0)
    binv = jnp.where(bg > 0, 1.0 / bg, 0.0)
    o_ref[...] = s * binv[:, None]


def _fuse_edge(p, bgp):
    return pl.pallas_call(
        _fuse_edge_body,
        grid=(NP // _BM,),
        in_specs=[
            pl.BlockSpec((NC, _BM, D), lambda i: (0, i, 0)),
            pl.BlockSpec((NW, _BM), lambda i: (0, i)),
        ],
        out_specs=pl.BlockSpec((_BM, D), lambda i: (i, 0)),
        out_shape=jax.ShapeDtypeStruct((NP, D), _f32),
    )(p, bgp)


def _fuse_node_mm_body(q_ref, dgp_ref, b_ref, w_ref, o_ref):
    s = q_ref[0] + q_ref[1]
    dg = jnp.sum(dgp_ref[...], axis=0)
    dinv = jnp.where(dg > 0, 1.0 / dg, 0.0)
    h = jnp.maximum(s * dinv[:, None] + b_ref[...], 0.0)
    o_ref[...] = jnp.dot(h, w_ref[...], preferred_element_type=_f32)


def _fuse_node_mm(q, dgp, b, w):
    return pl.pallas_call(
        _fuse_node_mm_body,
        grid=(NP // _BM,),
        in_specs=[
            pl.BlockSpec((NC, _BM, D), lambda i: (0, i, 0)),
            pl.BlockSpec((NW, _BM), lambda i: (0, i)),
            pl.BlockSpec((1, D), lambda i: (0, 0)),
            pl.BlockSpec((D, D), lambda i: (0, 0)),
        ],
        out_specs=pl.BlockSpec((_BM, D), lambda i: (i, 0)),
        out_shape=jax.ShapeDtypeStruct((NP, D), _f32),
    )(q, dgp, b.reshape(1, D), w)


def _fuse_node_body(q_ref, dgp_ref, b_ref, o_ref):
    s = q_ref[0] + q_ref[1]
    dg = jnp.sum(dgp_ref[...], axis=0)
    dinv = jnp.where(dg > 0, 1.0 / dg, 0.0)
    o_ref[...] = jnp.maximum(s * dinv[:, None] + b_ref[...], 0.0)


def _fuse_node(q, dgp, b):
    return pl.pallas_call(
        _fuse_node_body,
        grid=(NP // _BM,),
        in_specs=[
            pl.BlockSpec((NC, _BM, D), lambda i: (0, i, 0)),
            pl.BlockSpec((NW, _BM), lambda i: (0, i)),
            pl.BlockSpec((1, D), lambda i: (0, 0)),
        ],
        out_specs=pl.BlockSpec((_BM, D), lambda i: (i, 0)),
        out_shape=jax.ShapeDtypeStruct((NP, D), _f32),
    )(q, dgp, b.reshape(1, D))


def _to_col(a):
    return a.reshape(NP, NS, COLS).transpose(1, 0, 2)


def _from_col(p):
    return p.transpose(0, 2, 1, 3).reshape(NC, NP, D)


def kernel(x, hyperedge_index, hyperedge_weight, hyperedge_attr, batch,
           W1, b1, W2, b2, W3, b3):
    n_i = hyperedge_index[0].astype(_i32)
    e_i = hyperedge_index[1].astype(_i32)

    seg_shape = (NC, SEG_NSUP, SEG_SUP, SEG_K)
    gidx_n = n_i.reshape(seg_shape)
    sidx_e = e_i.reshape(seg_shape)
    gidx_e = e_i.reshape(seg_shape)
    sidx_n = n_i.reshape(seg_shape)

    dgp, bgp = _deg_kernel(n_i, e_i, hyperedge_weight)
    dgp = dgp.reshape(NW, NP)
    bgp = bgp.reshape(NW, NP)

    zeros = jnp.zeros((NP, COLS), _f32)
    xp = jnp.zeros((NP, D), _f32).at[:N].set(x)

    xw = _to_col(_mm(xp, W1))
    for (bcur, wnext) in ((b1, W2), (b2, W3)):
        p = _seg_kernel(xw, gidx_n, sidx_e, zeros)
        ef = _to_col(_fuse_edge(_from_col(p), bgp))
        q = _seg_kernel(ef, gidx_e, sidx_n, zeros)
        xw = _to_col(_fuse_node_mm(_from_col(q), dgp, bcur, wnext))
    p = _seg_kernel(xw, gidx_n, sidx_e, zeros)
    ef = _to_col(_fuse_edge(_from_col(p), bgp))
    q = _seg_kernel(ef, gidx_e, sidx_n, zeros)
    h = _fuse_node(_from_col(q), dgp, b3)
    return h[:N]


# speedup vs baseline: 1.5351x; 1.5351x over previous
import functools

import jax
import jax.numpy as jnp
from jax import lax
from jax.experimental import pallas as pl
from jax.experimental.pallas import tpu as pltpu
from jax.experimental.pallas import tpu_sc as plsc

N = 10000
E = 10000
NNZ = 320000
D = 128
NP = 10240
NC = 2
NS = 16
NW = NC * NS
PER_W = NNZ // NW
COLS = D // NS
SEG_K = 128
SEG_SUP = 10
SEG_NSUP = (NNZ // NC) // (SEG_K * SEG_SUP)

_f32 = jnp.float32
_i32 = jnp.int32

_MESH = plsc.VectorSubcoreMesh(core_axis_name="c", subcore_axis_name="s")


@functools.partial(
    pl.kernel,
    out_type=(
        jax.ShapeDtypeStruct((NW, NP // 16, 16), _f32),
        jax.ShapeDtypeStruct((NW, NP // 16, 16), _f32),
    ),
    mesh=_MESH,
    scratch_types=[
        pltpu.VMEM((PER_W,), _i32),
        pltpu.VMEM((PER_W,), _i32),
        pltpu.VMEM((E,), _f32),
        pltpu.VMEM((NP // 16, 16), _f32),
        pltpu.VMEM((NP // 16, 16), _f32),
    ],
    compiler_params=pltpu.CompilerParams(needs_layout_passes=False,
                                         use_tc_tiling_on_sc=False),
)
def _deg_kernel(nidx_hbm, eidx_hbm, hw_hbm, dg_out, bg_out,
                nidx_v, eidx_v, hw_v, dg_v, bg_v):
    cid = lax.axis_index("c")
    sid = lax.axis_index("s")
    wid = cid * NS + sid
    base = wid * PER_W
    pltpu.sync_copy(nidx_hbm.at[pl.ds(base, PER_W)], nidx_v)
    pltpu.sync_copy(eidx_hbm.at[pl.ds(base, PER_W)], eidx_v)
    pltpu.sync_copy(hw_hbm, hw_v)

    zeros16 = jnp.zeros((16,), _f32)

    def zero_body(i, carry):
        dg_v[i, :] = zeros16
        bg_v[i, :] = zeros16
        return carry

    lax.fori_loop(0, NP // 16, zero_body, 0)

    ones16 = jnp.ones((16,), _f32)

    def acc_body(i, carry):
        n16 = nidx_v[pl.ds(i * 16, 16)]
        e16 = eidx_v[pl.ds(i * 16, 16)]
        w16 = plsc.load_gather(hw_v, [e16])
        plsc.addupdate_scatter(dg_v, [n16 >> 4, n16 & 15], w16)
        plsc.addupdate_scatter(bg_v, [e16 >> 4, e16 & 15], ones16)
        return carry

    lax.fori_loop(0, PER_W // 16, acc_body, 0)

    pltpu.sync_copy(dg_v, dg_out.at[wid])
    pltpu.sync_copy(bg_v, bg_out.at[wid])


def _dyn_gather(v, idx):
    dn = lax.GatherDimensionNumbers(offset_dims=(), collapsed_slice_dims=(0,),
                                    start_index_map=(0,))
    return lax.gather(v, idx[:, None], dn, (1,),
                      mode=lax.GatherScatterMode.PROMISE_IN_BOUNDS)


@functools.partial(
    pl.kernel,
    out_type=jax.ShapeDtypeStruct((NC, NS, NP * COLS), _f32),
    mesh=_MESH,
    scratch_types=[
        pltpu.VMEM((SEG_SUP, SEG_K), _i32),
        pltpu.VMEM((SEG_SUP, SEG_K), _i32),
        pltpu.VMEM((SEG_K, COLS), _f32),
        pltpu.VMEM((SEG_K, COLS), _f32),
        pltpu.VMEM((NP * COLS,), _f32),
        pltpu.SemaphoreType.DMA,
        pltpu.SemaphoreType.DMA,
    ],
    compiler_params=pltpu.CompilerParams(needs_layout_passes=False,
                                         use_tc_tiling_on_sc=False),
)
def _seg_kernel(feat_hbm, gidx_hbm, sidx8_hbm, zeros_hbm, out_hbm,
                gidx_v, sidx_v, rows_a, rows_b, acc_v, sem_a, sem_b):
    cid = lax.axis_index("c")
    sid = lax.axis_index("s")
    pltpu.sync_copy(zeros_hbm, acc_v)

    lane = lax.iota(_i32, 16)
    col_sel = lane & 7
    half = lane >> 3

    def accumulate(buf, j):
        jvec = jnp.zeros((16,), _i32) + j

        @plsc.parallel_loop(0, SEG_K // 2, unroll=8)
        def _(i):
            pair_sel = half + 2 * i
            sel8 = plsc.load_gather(sidx_v, [jvec, pair_sel])
            vals = plsc.load_gather(buf, [pair_sel, col_sel])
            plsc.addupdate_scatter(acc_v, [sel8 + col_sel], vals)

    def pair_body(s, p, carry):
        j = p * 2
        da = pltpu.async_copy(feat_hbm.at[sid].at[gidx_v.at[j]],
                              rows_a, sem_a)
        db = pltpu.async_copy(feat_hbm.at[sid].at[gidx_v.at[j + 1]],
                              rows_b, sem_b)
        da.wait()
        accumulate(rows_a, j)
        db.wait()
        accumulate(rows_b, j + 1)
        return carry

    def sup_body(s, carry):
        pltpu.sync_copy(gidx_hbm.at[cid, s], gidx_v)
        pltpu.sync_copy(sidx8_hbm.at[cid, s], sidx_v)
        return lax.fori_loop(0, SEG_SUP // 2,
                             functools.partial(pair_body, s), carry)

    lax.fori_loop(0, SEG_NSUP, sup_body, 0)
    pltpu.sync_copy(acc_v, out_hbm.at[cid, sid])


_BM = 2048


def _mm_body(x_ref, w_ref, o_ref):
    o_ref[...] = jnp.dot(x_ref[...], w_ref[...], preferred_element_type=_f32)


def _mm(xp, w):
    return pl.pallas_call(
        _mm_body,
        grid=(NP // _BM,),
        in_specs=[
            pl.BlockSpec((_BM, D), lambda i: (i, 0)),
            pl.BlockSpec((D, D), lambda i: (0, 0)),
        ],
        out_specs=pl.BlockSpec((_BM, D), lambda i: (i, 0)),
        out_shape=jax.ShapeDtypeStruct((NP, D), _f32),
    )(xp, w)


def _fuse_edge_body(p_ref, bgp_ref, o_ref):
    s = p_ref[0] + p_ref[1]
    bg = jnp.sum(bgp_ref[...], axis=0)
    binv = jnp.where(bg > 0, 1.0 / bg, 0.0)
    o_ref[...] = s * binv[:, None]


def _fuse_edge(p, bgp):
    return pl.pallas_call(
        _fuse_edge_body,
        grid=(NP // _BM,),
        in_specs=[
            pl.BlockSpec((NC, _BM, D), lambda i: (0, i, 0)),
            pl.BlockSpec((NW, _BM), lambda i: (0, i)),
        ],
        out_specs=pl.BlockSpec((_BM, D), lambda i: (i, 0)),
        out_shape=jax.ShapeDtypeStruct((NP, D), _f32),
    )(p, bgp)


def _fuse_node_mm_body(q_ref, dgp_ref, b_ref, w_ref, o_ref):
    s = q_ref[0] + q_ref[1]
    dg = jnp.sum(dgp_ref[...], axis=0)
    dinv = jnp.where(dg > 0, 1.0 / dg, 0.0)
    h = jnp.maximum(s * dinv[:, None] + b_ref[...], 0.0)
    o_ref[...] = jnp.dot(h, w_ref[...], preferred_element_type=_f32)


def _fuse_node_mm(q, dgp, b, w):
    return pl.pallas_call(
        _fuse_node_mm_body,
        grid=(NP // _BM,),
        in_specs=[
            pl.BlockSpec((NC, _BM, D), lambda i: (0, i, 0)),
            pl.BlockSpec((NW, _BM), lambda i: (0, i)),
            pl.BlockSpec((1, D), lambda i: (0, 0)),
            pl.BlockSpec((D, D), lambda i: (0, 0)),
        ],
        out_specs=pl.BlockSpec((_BM, D), lambda i: (i, 0)),
        out_shape=jax.ShapeDtypeStruct((NP, D), _f32),
    )(q, dgp, b.reshape(1, D), w)


def _fuse_node_body(q_ref, dgp_ref, b_ref, o_ref):
    s = q_ref[0] + q_ref[1]
    dg = jnp.sum(dgp_ref[...], axis=0)
    dinv = jnp.where(dg > 0, 1.0 / dg, 0.0)
    o_ref[...] = jnp.maximum(s * dinv[:, None] + b_ref[...], 0.0)


def _fuse_node(q, dgp, b):
    return pl.pallas_call(
        _fuse_node_body,
        grid=(NP // _BM,),
        in_specs=[
            pl.BlockSpec((NC, _BM, D), lambda i: (0, i, 0)),
            pl.BlockSpec((NW, _BM), lambda i: (0, i)),
            pl.BlockSpec((1, D), lambda i: (0, 0)),
        ],
        out_specs=pl.BlockSpec((_BM, D), lambda i: (i, 0)),
        out_shape=jax.ShapeDtypeStruct((NP, D), _f32),
    )(q, dgp, b.reshape(1, D))


def _to_col(a):
    return a.reshape(NP, NS, COLS).transpose(1, 0, 2)


def _from_col(p):
    return p.reshape(NC, NS, NP, COLS).transpose(0, 2, 1, 3).reshape(NC, NP, D)


def kernel(x, hyperedge_index, hyperedge_weight, hyperedge_attr, batch,
           W1, b1, W2, b2, W3, b3):
    n_i = hyperedge_index[0].astype(_i32)
    e_i = hyperedge_index[1].astype(_i32)

    seg_shape = (NC, SEG_NSUP, SEG_SUP, SEG_K)
    gidx_n = n_i.reshape(seg_shape)
    sidx_e = (e_i * COLS).reshape(seg_shape)
    gidx_e = e_i.reshape(seg_shape)
    sidx_n = (n_i * COLS).reshape(seg_shape)

    dgp, bgp = _deg_kernel(n_i, e_i, hyperedge_weight)
    dgp = dgp.reshape(NW, NP)
    bgp = bgp.reshape(NW, NP)

    zeros = jnp.zeros((NP * COLS,), _f32)
    xp = jnp.zeros((NP, D), _f32).at[:N].set(x)

    xw = _to_col(_mm(xp, W1))
    for (bcur, wnext) in ((b1, W2), (b2, W3)):
        p = _seg_kernel(xw, gidx_n, sidx_e, zeros)
        ef = _to_col(_fuse_edge(_from_col(p), bgp))
        q = _seg_kernel(ef, gidx_e, sidx_n, zeros)
        xw = _to_col(_fuse_node_mm(_from_col(q), dgp, bcur, wnext))
    p = _seg_kernel(xw, gidx_n, sidx_e, zeros)
    ef = _to_col(_fuse_edge(_from_col(p), bgp))
    q = _seg_kernel(ef, gidx_e, sidx_n, zeros)
    h = _fuse_node(_from_col(q), dgp, b3)
    return h[:N]


# speedup vs baseline: 2.5330x; 1.6500x over previous
import functools

import jax
import jax.numpy as jnp
from jax import lax
from jax.experimental import pallas as pl
from jax.experimental.pallas import tpu as pltpu
from jax.experimental.pallas import tpu_sc as plsc

N = 10000
E = 10000
NNZ = 320000
D = 128
NP = 10240
NC = 2
NS = 16
NW = NC * NS
PER_W = NNZ // NW
COLS = D // NS
SEG_K = 128
SEG_SUP = 10
SEG_NSUP = (NNZ // NC) // (SEG_K * SEG_SUP)

_f32 = jnp.float32
_i32 = jnp.int32

_MESH = plsc.VectorSubcoreMesh(core_axis_name="c", subcore_axis_name="s")


@functools.partial(
    pl.kernel,
    out_type=(
        jax.ShapeDtypeStruct((NW, NP // 16, 16), _f32),
        jax.ShapeDtypeStruct((NW, NP // 16, 16), _f32),
    ),
    mesh=_MESH,
    scratch_types=[
        pltpu.VMEM((PER_W,), _i32),
        pltpu.VMEM((PER_W,), _i32),
        pltpu.VMEM((E,), _f32),
        pltpu.VMEM((NP // 16, 16), _f32),
        pltpu.VMEM((NP // 16, 16), _f32),
    ],
    compiler_params=pltpu.CompilerParams(needs_layout_passes=False,
                                         use_tc_tiling_on_sc=False),
)
def _deg_kernel(nidx_hbm, eidx_hbm, hw_hbm, dg_out, bg_out,
                nidx_v, eidx_v, hw_v, dg_v, bg_v):
    cid = lax.axis_index("c")
    sid = lax.axis_index("s")
    wid = cid * NS + sid
    base = wid * PER_W
    pltpu.sync_copy(nidx_hbm.at[pl.ds(base, PER_W)], nidx_v)
    pltpu.sync_copy(eidx_hbm.at[pl.ds(base, PER_W)], eidx_v)
    pltpu.sync_copy(hw_hbm, hw_v)

    zeros16 = jnp.zeros((16,), _f32)

    def zero_body(i, carry):
        dg_v[i, :] = zeros16
        bg_v[i, :] = zeros16
        return carry

    lax.fori_loop(0, NP // 16, zero_body, 0)

    ones16 = jnp.ones((16,), _f32)

    def acc_body(i, carry):
        n16 = nidx_v[pl.ds(i * 16, 16)]
        e16 = eidx_v[pl.ds(i * 16, 16)]
        w16 = plsc.load_gather(hw_v, [e16])
        plsc.addupdate_scatter(dg_v, [n16 >> 4, n16 & 15], w16)
        plsc.addupdate_scatter(bg_v, [e16 >> 4, e16 & 15], ones16)
        return carry

    lax.fori_loop(0, PER_W // 16, acc_body, 0)

    pltpu.sync_copy(dg_v, dg_out.at[wid])
    pltpu.sync_copy(bg_v, bg_out.at[wid])


def _dyn_gather(v, idx):
    dn = lax.GatherDimensionNumbers(offset_dims=(), collapsed_slice_dims=(0,),
                                    start_index_map=(0,))
    return lax.gather(v, idx[:, None], dn, (1,),
                      mode=lax.GatherScatterMode.PROMISE_IN_BOUNDS)


_IDXW = SEG_SUP * SEG_K


@functools.partial(
    pl.kernel,
    out_type=jax.ShapeDtypeStruct((NC, NS, NP * COLS), _f32),
    mesh=_MESH,
    scratch_types=(
        [pltpu.VMEM((2 * SEG_SUP, SEG_K), _i32),
         pltpu.VMEM((2 * _IDXW,), _i32),
         pltpu.VMEM((NP * COLS,), _f32)]
        + [pltpu.VMEM((SEG_K, COLS), _f32) for _ in range(SEG_SUP)]
        + [pltpu.SemaphoreType.DMA for _ in range(SEG_SUP + 2)]
    ),
    compiler_params=pltpu.CompilerParams(needs_layout_passes=False,
                                         use_tc_tiling_on_sc=False),
)
def _seg_kernel(feat_hbm, gidx_hbm, sidx8_hbm, zeros_hbm, out_hbm,
                gidx_v, sidx_v, acc_v, *bufs_and_sems):
    bufs = bufs_and_sems[:SEG_SUP]
    sems = bufs_and_sems[SEG_SUP:2 * SEG_SUP]
    sem_gi, sem_si = bufs_and_sems[2 * SEG_SUP:]
    cid = lax.axis_index("c")
    sid = lax.axis_index("s")
    feat = feat_hbm.at[sid]
    pltpu.sync_copy(zeros_hbm, acc_v)

    lane = lax.iota(_i32, 16)
    col_sel = lane & 7
    half = lane >> 3

    def accumulate(buf, base):
        @plsc.parallel_loop(0, SEG_K // 16, unroll=2)
        def _(g2):
            sidx16 = sidx_v[pl.ds(base + g2 * 16, 16)]
            for g in range(8):
                sel8 = _dyn_gather(sidx16, half + 2 * g)
                vals = plsc.load_gather(
                    bufs[buf], [half + (g2 * 16 + 2 * g), col_sel])
                plsc.addupdate_scatter(acc_v, [sel8 + col_sel], vals)

    def stage_idx(block, slot):
        pltpu.async_copy(gidx_hbm.at[cid, block],
                         gidx_v.at[pl.ds(slot * SEG_SUP, SEG_SUP)],
                         sem_gi)
        pltpu.async_copy(sidx8_hbm.at[cid, block],
                         sidx_v.at[pl.ds(slot * _IDXW, _IDXW)],
                         sem_si)

    pltpu.sync_copy(gidx_hbm.at[cid, 0],
                    gidx_v.at[pl.ds(0, SEG_SUP)])
    pltpu.sync_copy(sidx8_hbm.at[cid, 0],
                    sidx_v.at[pl.ds(0, _IDXW)])
    stage_idx(1, 1)
    for k in range(SEG_SUP):
        pltpu.async_copy(feat.at[gidx_v.at[k]], bufs[k], sems[k])

    def body(s, carry):
        par = s & 1
        nxt = 1 - par
        pltpu.make_async_copy(gidx_hbm.at[cid, s],
                              gidx_v.at[pl.ds(0, SEG_SUP)], sem_gi).wait()
        pltpu.make_async_copy(sidx8_hbm.at[cid, s],
                              sidx_v.at[pl.ds(0, _IDXW)], sem_si).wait()
        for k in range(SEG_SUP):
            pltpu.make_async_copy(feat.at[gidx_v.at[k]], bufs[k],
                                  sems[k]).wait()
            accumulate(k, par * _IDXW + k * SEG_K)
            pltpu.async_copy(feat.at[gidx_v.at[nxt * SEG_SUP + k]],
                             bufs[k], sems[k])
        stage_idx(s + 2, par)
        return carry

    lax.fori_loop(0, SEG_NSUP, body, 0)
    for k in range(SEG_SUP):
        pltpu.make_async_copy(feat.at[gidx_v.at[k]], bufs[k], sems[k]).wait()
    pltpu.make_async_copy(gidx_hbm.at[cid, 0],
                          gidx_v.at[pl.ds(0, SEG_SUP)], sem_gi).wait()
    pltpu.make_async_copy(sidx8_hbm.at[cid, 0],
                          sidx_v.at[pl.ds(0, _IDXW)], sem_si).wait()
    pltpu.sync_copy(acc_v, out_hbm.at[cid, sid])


_BM = 2048


def _mm_body(x_ref, w_ref, o_ref):
    o_ref[...] = jnp.dot(x_ref[...], w_ref[...], preferred_element_type=_f32)


def _mm(xp, w):
    return pl.pallas_call(
        _mm_body,
        grid=(NP // _BM,),
        in_specs=[
            pl.BlockSpec((_BM, D), lambda i: (i, 0)),
            pl.BlockSpec((D, D), lambda i: (0, 0)),
        ],
        out_specs=pl.BlockSpec((_BM, D), lambda i: (i, 0)),
        out_shape=jax.ShapeDtypeStruct((NP, D), _f32),
    )(xp, w)


def _fuse_edge_body(p_ref, bgp_ref, o_ref):
    s = p_ref[0] + p_ref[1]
    bg = jnp.sum(bgp_ref[...], axis=0)
    binv = jnp.where(bg > 0, 1.0 / bg, 0.0)
    o_ref[...] = s * binv[:, None]


def _fuse_edge(p, bgp):
    return pl.pallas_call(
        _fuse_edge_body,
        grid=(NP // _BM,),
        in_specs=[
            pl.BlockSpec((NC, _BM, D), lambda i: (0, i, 0)),
            pl.BlockSpec((NW, _BM), lambda i: (0, i)),
        ],
        out_specs=pl.BlockSpec((_BM, D), lambda i: (i, 0)),
        out_shape=jax.ShapeDtypeStruct((NP, D), _f32),
    )(p, bgp)


def _fuse_node_mm_body(q_ref, dgp_ref, b_ref, w_ref, o_ref):
    s = q_ref[0] + q_ref[1]
    dg = jnp.sum(dgp_ref[...], axis=0)
    dinv = jnp.where(dg > 0, 1.0 / dg, 0.0)
    h = jnp.maximum(s * dinv[:, None] + b_ref[...], 0.0)
    o_ref[...] = jnp.dot(h, w_ref[...], preferred_element_type=_f32)


def _fuse_node_mm(q, dgp, b, w):
    return pl.pallas_call(
        _fuse_node_mm_body,
        grid=(NP // _BM,),
        in_specs=[
            pl.BlockSpec((NC, _BM, D), lambda i: (0, i, 0)),
            pl.BlockSpec((NW, _BM), lambda i: (0, i)),
            pl.BlockSpec((1, D), lambda i: (0, 0)),
            pl.BlockSpec((D, D), lambda i: (0, 0)),
        ],
        out_specs=pl.BlockSpec((_BM, D), lambda i: (i, 0)),
        out_shape=jax.ShapeDtypeStruct((NP, D), _f32),
    )(q, dgp, b.reshape(1, D), w)


def _fuse_node_body(q_ref, dgp_ref, b_ref, o_ref):
    s = q_ref[0] + q_ref[1]
    dg = jnp.sum(dgp_ref[...], axis=0)
    dinv = jnp.where(dg > 0, 1.0 / dg, 0.0)
    o_ref[...] = jnp.maximum(s * dinv[:, None] + b_ref[...], 0.0)


def _fuse_node(q, dgp, b):
    return pl.pallas_call(
        _fuse_node_body,
        grid=(NP // _BM,),
        in_specs=[
            pl.BlockSpec((NC, _BM, D), lambda i: (0, i, 0)),
            pl.BlockSpec((NW, _BM), lambda i: (0, i)),
            pl.BlockSpec((1, D), lambda i: (0, 0)),
        ],
        out_specs=pl.BlockSpec((_BM, D), lambda i: (i, 0)),
        out_shape=jax.ShapeDtypeStruct((NP, D), _f32),
    )(q, dgp, b.reshape(1, D))


def _to_col(a):
    return a.reshape(NP, NS, COLS).transpose(1, 0, 2)


def _from_col(p):
    return p.reshape(NC, NS, NP, COLS).transpose(0, 2, 1, 3).reshape(NC, NP, D)


def kernel(x, hyperedge_index, hyperedge_weight, hyperedge_attr, batch,
           W1, b1, W2, b2, W3, b3):
    n_i = hyperedge_index[0].astype(_i32)
    e_i = hyperedge_index[1].astype(_i32)

    def _gidx(ix):
        a = ix.reshape(NC, SEG_NSUP, SEG_SUP, SEG_K)
        return jnp.concatenate(
            [a, jnp.zeros((NC, 2, SEG_SUP, SEG_K), _i32)], axis=1)

    def _sidx(ix):
        a = (ix * COLS).reshape(NC, SEG_NSUP, _IDXW)
        return jnp.concatenate([a, jnp.zeros((NC, 2, _IDXW), _i32)], axis=1)

    gidx_n = _gidx(n_i)
    sidx_e = _sidx(e_i)
    gidx_e = _gidx(e_i)
    sidx_n = _sidx(n_i)

    dgp, bgp = _deg_kernel(n_i, e_i, hyperedge_weight)
    dgp = dgp.reshape(NW, NP)
    bgp = bgp.reshape(NW, NP)

    zeros = jnp.zeros((NP * COLS,), _f32)
    xp = jnp.zeros((NP, D), _f32).at[:N].set(x)

    xw = _to_col(_mm(xp, W1))
    for (bcur, wnext) in ((b1, W2), (b2, W3)):
        p = _seg_kernel(xw, gidx_n, sidx_e, zeros)
        ef = _to_col(_fuse_edge(_from_col(p), bgp))
        q = _seg_kernel(ef, gidx_e, sidx_n, zeros)
        xw = _to_col(_fuse_node_mm(_from_col(q), dgp, bcur, wnext))
    p = _seg_kernel(xw, gidx_n, sidx_e, zeros)
    ef = _to_col(_fuse_edge(_from_col(p), bgp))
    q = _seg_kernel(ef, gidx_e, sidx_n, zeros)
    h = _fuse_node(_from_col(q), dgp, b3)
    return h[:N]
